# baseline (speedup 1.0000x reference)
import functools

import jax
import jax.numpy as jnp
from jax import lax
from jax.experimental import pallas as pl
from jax.experimental.pallas import tpu as pltpu
from jax.experimental.pallas import tpu_sc as plsc

N = 4096
EDIM = 128
WFEAT = 128
SLOPE = 0.01
BLK = 512
NBLK = N // BLK
CW = 2048
NCOL = N // CW


def _sc_gather(emb, idx):
    info = plsc.get_sparse_core_info()
    nc, ns = info.num_cores, info.num_subcores
    nw = nc * ns
    b_per_w = N // nw
    mesh = plsc.VectorSubcoreMesh(core_axis_name="c", subcore_axis_name="s")

    @functools.partial(
        pl.kernel,
        mesh=mesh,
        out_type=jax.ShapeDtypeStruct((N, EDIM), jnp.float32),
        scratch_types=[
            pltpu.VMEM((b_per_w,), jnp.int32),
            pltpu.VMEM((b_per_w, EDIM), jnp.float32),
            pltpu.SemaphoreType.DMA,
        ],
    )
    def gather_k(table_hbm, idx_hbm, out_hbm, idx_v, rows_v, sem):
        wid = lax.axis_index("s") * nc + lax.axis_index("c")
        base = wid * b_per_w
        pltpu.sync_copy(idx_hbm.at[pl.ds(base, b_per_w)], idx_v)
        pltpu.async_copy(table_hbm.at[idx_v], rows_v, sem).wait()
        pltpu.sync_copy(rows_v, out_hbm.at[pl.ds(base, b_per_w)])

    return gather_k(emb, idx)


def _phase_init(x, W_ref, a2_ref, h_ref, hb_ref, f2_ref, m2_ref):
    h = jnp.dot(x, W_ref[...], preferred_element_type=jnp.float32)
    h_ref[...] = h
    hb_ref[...] = h.astype(jnp.bfloat16)
    f2row = lax.dot_general(a2_ref[...], h, (((1,), (1,)), ((), ())),
                            preferred_element_type=jnp.float32)
    f2_ref[...] = f2row
    m2_ref[...] = jnp.max(f2row, axis=1, keepdims=True)


def _att_block(j, a1_ref, adj_ref, h_ref, hb_ref, f2_ref, m2_ref, att_ref):
    h_blk = h_ref[pl.ds(j * BLK, BLK), :]
    f1 = jnp.dot(h_blk, a1_ref[...], preferred_element_type=jnp.float32)
    m = f1 + m2_ref[...]
    m = jnp.maximum(m, SLOPE * m)
    s = jnp.zeros((BLK, 1), jnp.float32)
    for c in range(NCOL):
        f2h = f2_ref[:, pl.ds(c * CW, CW)]
        e = f1 + f2h
        e = jnp.maximum(e, SLOPE * e)
        p = jnp.where(adj_ref[:, pl.ds(c * CW, CW)] != 0,
                      jnp.exp(e - m), 0.0)
        att_ref[:, pl.ds(c * CW, CW)] = p
        ones = f2h * 0.0 + 1.0
        s = s + lax.dot_general(p, ones, (((1,), (1,)), ((), ())),
                                preferred_element_type=jnp.float32)
    r = 1.0 / s
    o = jnp.dot(att_ref[...].astype(jnp.bfloat16), hb_ref[...],
                preferred_element_type=jnp.float32) * r
    att_ref[...] = att_ref[...] * r
    return jnp.where(o > 0, o, jnp.exp(o) - 1.0)


def _fused_body(x_ref, Ws_ref, a1s_ref, a2s_ref, Wd_ref, a1d_ref, a2d_ref,
                adj0_ref, adj1_ref, satt_ref, datt_ref, dsum_ref,
                h_ref, hb_ref, f2_ref, m2_ref, x2_ref):
    i = pl.program_id(0)

    @pl.when(i == 0)
    def _init1():
        _phase_init(x_ref[...], Ws_ref, a2s_ref, h_ref, hb_ref, f2_ref,
                    m2_ref)

    @pl.when(i == NBLK)
    def _init2():
        _phase_init(x2_ref[...], Wd_ref, a2d_ref, h_ref, hb_ref, f2_ref,
                    m2_ref)

    @pl.when(i < NBLK)
    def _layer1():
        out = _att_block(i, a1s_ref, adj0_ref, h_ref, hb_ref, f2_ref,
                         m2_ref, satt_ref)
        x2_ref[pl.ds(i * BLK, BLK), :] = out

    @pl.when(i >= NBLK)
    def _layer2():
        j = i - NBLK
        out = _att_block(j, a1d_ref, adj1_ref, h_ref, hb_ref, f2_ref,
                         m2_ref, datt_ref)
        doc = out + x2_ref[pl.ds(j * BLK, BLK), :]

        @pl.when(j == 0)
        def _zero():
            dsum_ref[...] = jnp.zeros_like(dsum_ref)

        dsum_ref[...] = dsum_ref[...] + jnp.sum(doc, axis=0, keepdims=True)

        @pl.when(j == NBLK - 1)
        def _scale():
            dsum_ref[...] = dsum_ref[...] * (1.0 / N)


def _fused_gat(x, adj0, adj1, W_s, a_s, W_d, a_d):
    a1s = a_s[:WFEAT].reshape(WFEAT, 1)
    a2s = a_s[WFEAT:].reshape(1, WFEAT)
    a1d = a_d[:WFEAT].reshape(WFEAT, 1)
    a2d = a_d[WFEAT:].reshape(1, WFEAT)
    last = NBLK - 1
    satt, datt, dsum = pl.pallas_call(
        _fused_body,
        grid=(2 * NBLK,),
        in_specs=[
            pl.BlockSpec((N, EDIM), lambda i: (0, 0)),
            pl.BlockSpec((EDIM, WFEAT), lambda i: (0, 0)),
            pl.BlockSpec((WFEAT, 1), lambda i: (0, 0)),
            pl.BlockSpec((1, WFEAT), lambda i: (0, 0)),
            pl.BlockSpec((WFEAT, WFEAT), lambda i: (0, 0)),
            pl.BlockSpec((WFEAT, 1), lambda i: (0, 0)),
            pl.BlockSpec((1, WFEAT), lambda i: (0, 0)),
            pl.BlockSpec((BLK, N), lambda i: (jnp.minimum(i, last), 0)),
            pl.BlockSpec((BLK, N), lambda i: (jnp.maximum(i - NBLK, 0), 0)),
        ],
        out_specs=[
            pl.BlockSpec((BLK, N), lambda i: (jnp.minimum(i, last), 0)),
            pl.BlockSpec((BLK, N), lambda i: (jnp.maximum(i - NBLK, 0), 0)),
            pl.BlockSpec((1, WFEAT), lambda i: (0, 0)),
        ],
        out_shape=[
            jax.ShapeDtypeStruct((N, N), jnp.float32),
            jax.ShapeDtypeStruct((N, N), jnp.float32),
            jax.ShapeDtypeStruct((1, WFEAT), jnp.float32),
        ],
        scratch_shapes=[
            pltpu.VMEM((N, WFEAT), jnp.float32),
            pltpu.VMEM((N, WFEAT), jnp.bfloat16),
            pltpu.VMEM((1, N), jnp.float32),
            pltpu.VMEM((1, 1), jnp.float32),
            pltpu.VMEM((N, WFEAT), jnp.float32),
        ],
        compiler_params=pltpu.CompilerParams(
            dimension_semantics=("arbitrary",)),
    )(x, W_s, a1s, a2s, W_d, a1d, a2d,
      adj0.view(jnp.int8), adj1.view(jnp.int8))
    return satt, datt, dsum


def kernel(inDoc, adj0, adj1, emb, W_s, a_s, W_d, a_d):
    words = _sc_gather(emb, inDoc.astype(jnp.int32))
    sattention, dattention, dsum = _fused_gat(
        words, adj0, adj1, W_s, a_s, W_d, a_d)
    return (dsum[0], sattention, dattention)


# speedup vs baseline: 1.0591x; 1.0591x over previous
import functools

import jax
import jax.numpy as jnp
from jax import lax
from jax.experimental import pallas as pl
from jax.experimental.pallas import tpu as pltpu
from jax.experimental.pallas import tpu_sc as plsc

N = 4096
EDIM = 128
WFEAT = 128
SLOPE = 0.01
BLK = 512
NBLK = N // BLK
CW = 1024
NCOL = N // CW


def _sc_gather(emb, idx):
    info = plsc.get_sparse_core_info()
    nc, ns = info.num_cores, info.num_subcores
    nw = nc * ns
    b_per_w = N // nw
    mesh = plsc.VectorSubcoreMesh(core_axis_name="c", subcore_axis_name="s")

    @functools.partial(
        pl.kernel,
        mesh=mesh,
        out_type=jax.ShapeDtypeStruct((N, EDIM), jnp.float32),
        scratch_types=[
            pltpu.VMEM((b_per_w,), jnp.int32),
            pltpu.VMEM((b_per_w, EDIM), jnp.float32),
            pltpu.SemaphoreType.DMA,
        ],
    )
    def gather_k(table_hbm, idx_hbm, out_hbm, idx_v, rows_v, sem):
        wid = lax.axis_index("s") * nc + lax.axis_index("c")
        base = wid * b_per_w
        pltpu.sync_copy(idx_hbm.at[pl.ds(base, b_per_w)], idx_v)
        pltpu.async_copy(table_hbm.at[idx_v], rows_v, sem).wait()
        pltpu.sync_copy(rows_v, out_hbm.at[pl.ds(base, b_per_w)])

    return gather_k(emb, idx)


def _phase_init(x, W_ref, a2_ref, h_ref, hb_ref, f2_ref, m2_ref):
    h = jnp.dot(x, W_ref[...], preferred_element_type=jnp.float32)
    h_ref[...] = h
    hb_ref[...] = h.astype(jnp.bfloat16)
    f2row = lax.dot_general(a2_ref[...], h, (((1,), (1,)), ((), ())),
                            preferred_element_type=jnp.float32)
    f2_ref[...] = f2row
    m2_ref[...] = jnp.max(f2row, axis=1, keepdims=True)


def _att_block(j, a1_ref, adj_ref, h_ref, hb_ref, f2_ref, m2_ref, att_ref):
    h_blk = h_ref[pl.ds(j * BLK, BLK), :]
    f1 = jnp.dot(h_blk, a1_ref[...], preferred_element_type=jnp.float32)
    m = f1 + m2_ref[...]
    m = jnp.maximum(m, SLOPE * m)
    f1m = f1 - m
    cm = (SLOPE - 1.0) * m
    s = jnp.zeros((BLK, 1), jnp.float32)
    for c in range(NCOL):
        f2h = f2_ref[:, pl.ds(c * CW, CW)]
        a = f1m + f2h
        t = jnp.maximum(a, SLOPE * a + cm)
        p = jnp.where(adj_ref[:, pl.ds(c * CW, CW)] != 0, jnp.exp2(t), 0.0)
        att_ref[:, pl.ds(c * CW, CW)] = p
        ones = f2h * 0.0 + 1.0
        s = s + lax.dot_general(p, ones, (((1,), (1,)), ((), ())),
                                preferred_element_type=jnp.float32)
    r = 1.0 / s
    o = jnp.dot(att_ref[...].astype(jnp.bfloat16), hb_ref[...],
                preferred_element_type=jnp.float32) * r
    att_ref[...] = att_ref[...] * r
    return jnp.where(o > 0, o, jnp.exp(o) - 1.0)


def _fused_body(x_ref, Ws_ref, a1s_ref, a2s_ref, Wd_ref, a1d_ref, a2d_ref,
                adj0_ref, adj1_ref, satt_ref, datt_ref, dsum_ref,
                h_ref, hb_ref, f2_ref, m2_ref, x2_ref):
    i = pl.program_id(0)

    @pl.when(i == 0)
    def _init1():
        _phase_init(x_ref[...], Ws_ref, a2s_ref, h_ref, hb_ref, f2_ref,
                    m2_ref)

    @pl.when(i == NBLK)
    def _init2():
        _phase_init(x2_ref[...], Wd_ref, a2d_ref, h_ref, hb_ref, f2_ref,
                    m2_ref)

    @pl.when(i < NBLK)
    def _layer1():
        out = _att_block(i, a1s_ref, adj0_ref, h_ref, hb_ref, f2_ref,
                         m2_ref, satt_ref)
        x2_ref[pl.ds(i * BLK, BLK), :] = out

    @pl.when(i >= NBLK)
    def _layer2():
        j = i - NBLK
        out = _att_block(j, a1d_ref, adj1_ref, h_ref, hb_ref, f2_ref,
                         m2_ref, datt_ref)
        doc = out + x2_ref[pl.ds(j * BLK, BLK), :]

        @pl.when(j == 0)
        def _zero():
            dsum_ref[...] = jnp.zeros_like(dsum_ref)

        dsum_ref[...] = dsum_ref[...] + jnp.sum(doc, axis=0, keepdims=True)

        @pl.when(j == NBLK - 1)
        def _scale():
            dsum_ref[...] = dsum_ref[...] * (1.0 / N)


def _fused_gat(x, adj0, adj1, W_s, a_s, W_d, a_d):
    LOG2E = 1.4426950408889634
    a1s = (a_s[:WFEAT] * LOG2E).reshape(WFEAT, 1)
    a2s = (a_s[WFEAT:] * LOG2E).reshape(1, WFEAT)
    a1d = (a_d[:WFEAT] * LOG2E).reshape(WFEAT, 1)
    a2d = (a_d[WFEAT:] * LOG2E).reshape(1, WFEAT)
    last = NBLK - 1
    satt, datt, dsum = pl.pallas_call(
        _fused_body,
        grid=(2 * NBLK,),
        in_specs=[
            pl.BlockSpec((N, EDIM), lambda i: (0, 0)),
            pl.BlockSpec((EDIM, WFEAT), lambda i: (0, 0)),
            pl.BlockSpec((WFEAT, 1), lambda i: (0, 0)),
            pl.BlockSpec((1, WFEAT), lambda i: (0, 0)),
            pl.BlockSpec((WFEAT, WFEAT), lambda i: (0, 0)),
            pl.BlockSpec((WFEAT, 1), lambda i: (0, 0)),
            pl.BlockSpec((1, WFEAT), lambda i: (0, 0)),
            pl.BlockSpec((BLK, N), lambda i: (jnp.minimum(i, last), 0)),
            pl.BlockSpec((BLK, N), lambda i: (jnp.maximum(i - NBLK, 0), 0)),
        ],
        out_specs=[
            pl.BlockSpec((BLK, N), lambda i: (jnp.minimum(i, last), 0)),
            pl.BlockSpec((BLK, N), lambda i: (jnp.maximum(i - NBLK, 0), 0)),
            pl.BlockSpec((1, WFEAT), lambda i: (0, 0)),
        ],
        out_shape=[
            jax.ShapeDtypeStruct((N, N), jnp.float32),
            jax.ShapeDtypeStruct((N, N), jnp.float32),
            jax.ShapeDtypeStruct((1, WFEAT), jnp.float32),
        ],
        scratch_shapes=[
            pltpu.VMEM((N, WFEAT), jnp.float32),
            pltpu.VMEM((N, WFEAT), jnp.bfloat16),
            pltpu.VMEM((1, N), jnp.float32),
            pltpu.VMEM((1, 1), jnp.float32),
            pltpu.VMEM((N, WFEAT), jnp.float32),
        ],
        compiler_params=pltpu.CompilerParams(
            dimension_semantics=("arbitrary",)),
    )(x, W_s, a1s, a2s, W_d, a1d, a2d,
      adj0.view(jnp.int8), adj1.view(jnp.int8))
    return satt, datt, dsum


def kernel(inDoc, adj0, adj1, emb, W_s, a_s, W_d, a_d):
    words = _sc_gather(emb, inDoc.astype(jnp.int32))
    sattention, dattention, dsum = _fused_gat(
        words, adj0, adj1, W_s, a_s, W_d, a_d)
    return (dsum[0], sattention, dattention)


# speedup vs baseline: 1.0823x; 1.0219x over previous
import functools

import jax
import jax.numpy as jnp
from jax import lax
from jax.experimental import pallas as pl
from jax.experimental.pallas import tpu as pltpu
from jax.experimental.pallas import tpu_sc as plsc

N = 4096
EDIM = 128
WFEAT = 128
SLOPE = 0.01
BLK = 512
NBLK = N // BLK
CW = 512
NCOL = N // CW


def _sc_gather(emb, idx):
    info = plsc.get_sparse_core_info()
    nc, ns = info.num_cores, info.num_subcores
    nw = nc * ns
    b_per_w = N // nw
    mesh = plsc.VectorSubcoreMesh(core_axis_name="c", subcore_axis_name="s")

    @functools.partial(
        pl.kernel,
        mesh=mesh,
        out_type=jax.ShapeDtypeStruct((N, EDIM), jnp.float32),
        scratch_types=[
            pltpu.VMEM((b_per_w,), jnp.int32),
            pltpu.VMEM((b_per_w, EDIM), jnp.float32),
            pltpu.SemaphoreType.DMA,
        ],
    )
    def gather_k(table_hbm, idx_hbm, out_hbm, idx_v, rows_v, sem):
        wid = lax.axis_index("s") * nc + lax.axis_index("c")
        base = wid * b_per_w
        pltpu.sync_copy(idx_hbm.at[pl.ds(base, b_per_w)], idx_v)
        pltpu.async_copy(table_hbm.at[idx_v], rows_v, sem).wait()
        pltpu.sync_copy(rows_v, out_hbm.at[pl.ds(base, b_per_w)])

    return gather_k(emb, idx)


def _phase_init(x, W_ref, a2_ref, h_ref, hb_ref, f2_ref, m2_ref):
    h = jnp.dot(x, W_ref[...], preferred_element_type=jnp.float32)
    h_ref[...] = h
    hb_ref[...] = h.astype(jnp.bfloat16)
    f2row = lax.dot_general(a2_ref[...], h, (((1,), (1,)), ((), ())),
                            preferred_element_type=jnp.float32)
    f2_ref[...] = f2row
    m2_ref[...] = jnp.max(f2row, axis=1, keepdims=True)


def _att_block(j, a1_ref, adj_ref, h_ref, hb_ref, f2_ref, m2_ref, att_ref):
    h_blk = h_ref[pl.ds(j * BLK, BLK), :]
    f1 = jnp.dot(h_blk, a1_ref[...], preferred_element_type=jnp.float32)
    m = f1 + m2_ref[...]
    m = jnp.maximum(m, SLOPE * m)
    f1m = f1 - m
    cm = (SLOPE - 1.0) * m
    s = jnp.zeros((BLK, 1), jnp.float32)
    for c in range(NCOL):
        f2h = f2_ref[:, pl.ds(c * CW, CW)]
        a = f1m + f2h
        t = jnp.maximum(a, SLOPE * a + cm)
        p = jnp.where(adj_ref[:, pl.ds(c * CW, CW)] != 0, jnp.exp2(t), 0.0)
        att_ref[:, pl.ds(c * CW, CW)] = p
        ones = f2h * 0.0 + 1.0
        s = s + lax.dot_general(p, ones, (((1,), (1,)), ((), ())),
                                preferred_element_type=jnp.float32)
    r = 1.0 / s
    o = jnp.dot(att_ref[...].astype(jnp.bfloat16), hb_ref[...],
                preferred_element_type=jnp.float32) * r
    att_ref[...] = att_ref[...] * r
    return jnp.where(o > 0, o, jnp.exp(o) - 1.0)


def _fused_body(x_ref, Ws_ref, a1s_ref, a2s_ref, Wd_ref, a1d_ref, a2d_ref,
                adj0_ref, adj1_ref, satt_ref, datt_ref, dsum_ref,
                h_ref, hb_ref, f2_ref, m2_ref, x2_ref):
    i = pl.program_id(0)

    @pl.when(i == 0)
    def _init1():
        _phase_init(x_ref[...], Ws_ref, a2s_ref, h_ref, hb_ref, f2_ref,
                    m2_ref)

    @pl.when(i == NBLK)
    def _init2():
        _phase_init(x2_ref[...], Wd_ref, a2d_ref, h_ref, hb_ref, f2_ref,
                    m2_ref)

    @pl.when(i < NBLK)
    def _layer1():
        out = _att_block(i, a1s_ref, adj0_ref, h_ref, hb_ref, f2_ref,
                         m2_ref, satt_ref)
        x2_ref[pl.ds(i * BLK, BLK), :] = out

    @pl.when(i >= NBLK)
    def _layer2():
        j = i - NBLK
        out = _att_block(j, a1d_ref, adj1_ref, h_ref, hb_ref, f2_ref,
                         m2_ref, datt_ref)
        doc = out + x2_ref[pl.ds(j * BLK, BLK), :]

        @pl.when(j == 0)
        def _zero():
            dsum_ref[...] = jnp.zeros_like(dsum_ref)

        dsum_ref[...] = dsum_ref[...] + jnp.sum(doc, axis=0, keepdims=True)

        @pl.when(j == NBLK - 1)
        def _scale():
            dsum_ref[...] = dsum_ref[...] * (1.0 / N)


def _fused_gat(x, adj0, adj1, W_s, a_s, W_d, a_d):
    LOG2E = 1.4426950408889634
    a1s = (a_s[:WFEAT] * LOG2E).reshape(WFEAT, 1)
    a2s = (a_s[WFEAT:] * LOG2E).reshape(1, WFEAT)
    a1d = (a_d[:WFEAT] * LOG2E).reshape(WFEAT, 1)
    a2d = (a_d[WFEAT:] * LOG2E).reshape(1, WFEAT)
    last = NBLK - 1
    satt, datt, dsum = pl.pallas_call(
        _fused_body,
        grid=(2 * NBLK,),
        in_specs=[
            pl.BlockSpec((N, EDIM), lambda i: (0, 0)),
            pl.BlockSpec((EDIM, WFEAT), lambda i: (0, 0)),
            pl.BlockSpec((WFEAT, 1), lambda i: (0, 0)),
            pl.BlockSpec((1, WFEAT), lambda i: (0, 0)),
            pl.BlockSpec((WFEAT, WFEAT), lambda i: (0, 0)),
            pl.BlockSpec((WFEAT, 1), lambda i: (0, 0)),
            pl.BlockSpec((1, WFEAT), lambda i: (0, 0)),
            pl.BlockSpec((BLK, N), lambda i: (jnp.minimum(i, last), 0)),
            pl.BlockSpec((BLK, N), lambda i: (jnp.maximum(i - NBLK, 0), 0)),
        ],
        out_specs=[
            pl.BlockSpec((BLK, N), lambda i: (jnp.minimum(i, last), 0)),
            pl.BlockSpec((BLK, N), lambda i: (jnp.maximum(i - NBLK, 0), 0)),
            pl.BlockSpec((1, WFEAT), lambda i: (0, 0)),
        ],
        out_shape=[
            jax.ShapeDtypeStruct((N, N), jnp.float32),
            jax.ShapeDtypeStruct((N, N), jnp.float32),
            jax.ShapeDtypeStruct((1, WFEAT), jnp.float32),
        ],
        scratch_shapes=[
            pltpu.VMEM((N, WFEAT), jnp.float32),
            pltpu.VMEM((N, WFEAT), jnp.bfloat16),
            pltpu.VMEM((1, N), jnp.float32),
            pltpu.VMEM((1, 1), jnp.float32),
            pltpu.VMEM((N, WFEAT), jnp.float32),
        ],
        compiler_params=pltpu.CompilerParams(
            dimension_semantics=("arbitrary",)),
    )(x, W_s, a1s, a2s, W_d, a1d, a2d,
      adj0.view(jnp.int8), adj1.view(jnp.int8))
    return satt, datt, dsum


def kernel(inDoc, adj0, adj1, emb, W_s, a_s, W_d, a_d):
    words = _sc_gather(emb, inDoc.astype(jnp.int32))
    sattention, dattention, dsum = _fused_gat(
        words, adj0, adj1, W_s, a_s, W_d, a_d)
    return (dsum[0], sattention, dattention)


# speedup vs baseline: 1.1067x; 1.0226x over previous
import functools

import jax
import jax.numpy as jnp
from jax import lax
from jax.experimental import pallas as pl
from jax.experimental.pallas import tpu as pltpu
from jax.experimental.pallas import tpu_sc as plsc

N = 4096
EDIM = 128
WFEAT = 128
SLOPE = 0.01
BLK = 512
NBLK = N // BLK
CW = 256
NCOL = N // CW


def _sc_gather(emb, idx):
    info = plsc.get_sparse_core_info()
    nc, ns = info.num_cores, info.num_subcores
    nw = nc * ns
    b_per_w = N // nw
    mesh = plsc.VectorSubcoreMesh(core_axis_name="c", subcore_axis_name="s")

    @functools.partial(
        pl.kernel,
        mesh=mesh,
        out_type=jax.ShapeDtypeStruct((N, EDIM), jnp.float32),
        scratch_types=[
            pltpu.VMEM((b_per_w,), jnp.int32),
            pltpu.VMEM((b_per_w, EDIM), jnp.float32),
            pltpu.SemaphoreType.DMA,
        ],
    )
    def gather_k(table_hbm, idx_hbm, out_hbm, idx_v, rows_v, sem):
        wid = lax.axis_index("s") * nc + lax.axis_index("c")
        base = wid * b_per_w
        pltpu.sync_copy(idx_hbm.at[pl.ds(base, b_per_w)], idx_v)
        pltpu.async_copy(table_hbm.at[idx_v], rows_v, sem).wait()
        pltpu.sync_copy(rows_v, out_hbm.at[pl.ds(base, b_per_w)])

    return gather_k(emb, idx)


def _phase_init(x, W_ref, a2_ref, h_ref, hb_ref, f2_ref, m2_ref):
    h = jnp.dot(x, W_ref[...], preferred_element_type=jnp.float32)
    h_ref[...] = h
    hb_ref[...] = h.astype(jnp.bfloat16)
    f2row = lax.dot_general(a2_ref[...], h, (((1,), (1,)), ((), ())),
                            preferred_element_type=jnp.float32)
    f2_ref[...] = f2row
    m2_ref[...] = jnp.max(f2row, axis=1, keepdims=True)


def _att_block(j, a1_ref, adj_ref, h_ref, hb_ref, f2_ref, m2_ref, att_ref):
    h_blk = h_ref[pl.ds(j * BLK, BLK), :]
    f1 = jnp.dot(h_blk, a1_ref[...], preferred_element_type=jnp.float32)
    m = f1 + m2_ref[...]
    m = jnp.maximum(m, SLOPE * m)
    f1m = f1 - m
    cm = (SLOPE - 1.0) * m
    s = jnp.zeros((BLK, 1), jnp.float32)
    for c in range(NCOL):
        f2h = f2_ref[:, pl.ds(c * CW, CW)]
        a = f1m + f2h
        t = jnp.maximum(a, SLOPE * a + cm)
        p = jnp.where(adj_ref[:, pl.ds(c * CW, CW)] != 0, jnp.exp2(t), 0.0)
        att_ref[:, pl.ds(c * CW, CW)] = p
        ones = f2h * 0.0 + 1.0
        s = s + lax.dot_general(p, ones, (((1,), (1,)), ((), ())),
                                preferred_element_type=jnp.float32)
    r = 1.0 / s
    o = jnp.dot(att_ref[...].astype(jnp.bfloat16), hb_ref[...],
                preferred_element_type=jnp.float32) * r
    att_ref[...] = att_ref[...] * r
    return jnp.where(o > 0, o, jnp.exp(o) - 1.0)


def _fused_body(x_ref, Ws_ref, a1s_ref, a2s_ref, Wd_ref, a1d_ref, a2d_ref,
                adj0_ref, adj1_ref, satt_ref, datt_ref, dsum_ref,
                h_ref, hb_ref, f2_ref, m2_ref, x2_ref):
    i = pl.program_id(0)

    @pl.when(i == 0)
    def _init1():
        _phase_init(x_ref[...], Ws_ref, a2s_ref, h_ref, hb_ref, f2_ref,
                    m2_ref)

    @pl.when(i == NBLK)
    def _init2():
        _phase_init(x2_ref[...], Wd_ref, a2d_ref, h_ref, hb_ref, f2_ref,
                    m2_ref)

    @pl.when(i < NBLK)
    def _layer1():
        out = _att_block(i, a1s_ref, adj0_ref, h_ref, hb_ref, f2_ref,
                         m2_ref, satt_ref)
        x2_ref[pl.ds(i * BLK, BLK), :] = out

    @pl.when(i >= NBLK)
    def _layer2():
        j = i - NBLK
        out = _att_block(j, a1d_ref, adj1_ref, h_ref, hb_ref, f2_ref,
                         m2_ref, datt_ref)
        doc = out + x2_ref[pl.ds(j * BLK, BLK), :]

        @pl.when(j == 0)
        def _zero():
            dsum_ref[...] = jnp.zeros_like(dsum_ref)

        dsum_ref[...] = dsum_ref[...] + jnp.sum(doc, axis=0, keepdims=True)

        @pl.when(j == NBLK - 1)
        def _scale():
            dsum_ref[...] = dsum_ref[...] * (1.0 / N)


def _fused_gat(x, adj0, adj1, W_s, a_s, W_d, a_d):
    LOG2E = 1.4426950408889634
    a1s = (a_s[:WFEAT] * LOG2E).reshape(WFEAT, 1)
    a2s = (a_s[WFEAT:] * LOG2E).reshape(1, WFEAT)
    a1d = (a_d[:WFEAT] * LOG2E).reshape(WFEAT, 1)
    a2d = (a_d[WFEAT:] * LOG2E).reshape(1, WFEAT)
    last = NBLK - 1
    satt, datt, dsum = pl.pallas_call(
        _fused_body,
        grid=(2 * NBLK,),
        in_specs=[
            pl.BlockSpec((N, EDIM), lambda i: (0, 0)),
            pl.BlockSpec((EDIM, WFEAT), lambda i: (0, 0)),
            pl.BlockSpec((WFEAT, 1), lambda i: (0, 0)),
            pl.BlockSpec((1, WFEAT), lambda i: (0, 0)),
            pl.BlockSpec((WFEAT, WFEAT), lambda i: (0, 0)),
            pl.BlockSpec((WFEAT, 1), lambda i: (0, 0)),
            pl.BlockSpec((1, WFEAT), lambda i: (0, 0)),
            pl.BlockSpec((BLK, N), lambda i: (jnp.minimum(i, last), 0)),
            pl.BlockSpec((BLK, N), lambda i: (jnp.maximum(i - NBLK, 0), 0)),
        ],
        out_specs=[
            pl.BlockSpec((BLK, N), lambda i: (jnp.minimum(i, last), 0)),
            pl.BlockSpec((BLK, N), lambda i: (jnp.maximum(i - NBLK, 0), 0)),
            pl.BlockSpec((1, WFEAT), lambda i: (0, 0)),
        ],
        out_shape=[
            jax.ShapeDtypeStruct((N, N), jnp.float32),
            jax.ShapeDtypeStruct((N, N), jnp.float32),
            jax.ShapeDtypeStruct((1, WFEAT), jnp.float32),
        ],
        scratch_shapes=[
            pltpu.VMEM((N, WFEAT), jnp.float32),
            pltpu.VMEM((N, WFEAT), jnp.bfloat16),
            pltpu.VMEM((1, N), jnp.float32),
            pltpu.VMEM((1, 1), jnp.float32),
            pltpu.VMEM((N, WFEAT), jnp.float32),
        ],
        compiler_params=pltpu.CompilerParams(
            dimension_semantics=("arbitrary",)),
    )(x, W_s, a1s, a2s, W_d, a1d, a2d,
      adj0.view(jnp.int8), adj1.view(jnp.int8))
    return satt, datt, dsum


def kernel(inDoc, adj0, adj1, emb, W_s, a_s, W_d, a_d):
    words = _sc_gather(emb, inDoc.astype(jnp.int32))
    sattention, dattention, dsum = _fused_gat(
        words, adj0, adj1, W_s, a_s, W_d, a_d)
    return (dsum[0], sattention, dattention)
